# baseline (speedup 1.0000x reference)
import functools

import jax
import jax.numpy as jnp
import numpy as np
from jax import lax
from jax.experimental import pallas as pl
from jax.experimental.pallas import tpu as pltpu


def _stats_kernel(n_list, x0, x1, x2, a0, b0, a1, b1, a2, b2):
    for n, x, a, b in ((n_list[0], x0, a0, b0),
                       (n_list[1], x1, a1, b1),
                       (n_list[2], x2, a2, b2)):
        xv = x[...]
        mu = jnp.mean(xv, axis=0, keepdims=True)
        var = jnp.sum((xv - mu) ** 2, axis=0, keepdims=True) / (n - 1)
        s = jnp.sqrt(var) + 1e-6
        alpha = (1.0 / np.sqrt(n - 1)) / s
        a[...] = alpha
        b[...] = mu * alpha


def _cross_kernel(al, bl, au, bu, xl, xu, bd, out_cross, out_l, out_u, p_acc):
    i = pl.program_id(0)
    j = pl.program_id(1)
    nj = pl.num_programs(1)

    yl = xl[...] * al[...] - bl[...]
    pp = lax.dot_general(bd[...], yl, (((1,), (0,)), ((), ())),
                         preferred_element_type=jnp.float32)

    @pl.when(j == 0)
    def _():
        p_acc[...] = pp

    @pl.when(j > 0)
    def _():
        p_acc[...] += pp

    @pl.when(i == 0)
    def _():
        out_l[j, :, :] = lax.dot_general(yl, yl, (((0,), (0,)), ((), ())),
                                         preferred_element_type=jnp.float32)

    @pl.when(j == 0)
    def _():
        yu = xu[...] * au[...] - bu[...]
        out_u[i, :, :] = lax.dot_general(yu, yu, (((0,), (0,)), ((), ())),
                                         preferred_element_type=jnp.float32)

    @pl.when(j == nj - 1)
    def _():
        yu = xu[...] * au[...] - bu[...]
        out_cross[i, :, :] = lax.dot_general(
            yu, p_acc[...], (((0,), (0,)), ((), ())),
            preferred_element_type=jnp.float32)


def _cross_call(alpha_l, beta_l, alpha_u, beta_u, Xl, Xu, Bdry, b):
    per_l = Xl.shape[0] // b
    per_u = Xu.shape[0] // b
    d = Xl.shape[1]
    out_sh = jax.ShapeDtypeStruct((b, d, d), jnp.float32)
    stat_spec = pl.BlockSpec((1, d), lambda i, j: (0, 0))
    corr_spec = pl.BlockSpec((b, d, d), lambda i, j: (0, 0, 0))
    return pl.pallas_call(
        _cross_kernel,
        grid=(b, b),
        in_specs=[
            stat_spec, stat_spec, stat_spec, stat_spec,
            pl.BlockSpec((per_l, d), lambda i, j: (j, 0)),
            pl.BlockSpec((per_u, d), lambda i, j: (i, 0)),
            pl.BlockSpec((per_u, per_l), lambda i, j: (i, j)),
        ],
        out_specs=[corr_spec, corr_spec, corr_spec],
        out_shape=[out_sh, out_sh, out_sh],
        scratch_shapes=[pltpu.VMEM((per_u, d), jnp.float32)],
        compiler_params=pltpu.CompilerParams(
            dimension_semantics=("arbitrary", "arbitrary")),
    )(alpha_l, beta_l, alpha_u, beta_u, Xl, Xu, Bdry)


def kernel(X0, X1, X2, D2B1TD1inv, B2TD2inv, num_nodes, num_edges,
           num_triangles):
    b = len(num_nodes)
    d = X0.shape[1]
    n0, n1, n2 = X0.shape[0], X1.shape[0], X2.shape[0]

    stat_sh = jax.ShapeDtypeStruct((1, d), jnp.float32)
    a0, b0, a1, b1, a2, b2 = pl.pallas_call(
        functools.partial(_stats_kernel, (n0, n1, n2)),
        out_shape=[stat_sh] * 6,
    )(X0, X1, X2)

    X01corr, X0corr, X1corr = _cross_call(a0, b0, a1, b1, X0, X1,
                                          D2B1TD1inv, b)
    X12corr, _, X2corr = _cross_call(a1, b1, a2, b2, X1, X2, B2TD2inv, b)

    return (X0corr, X1corr, X2corr, X01corr, X12corr)
